# initial kernel scaffold (unmeasured)
import jax
import jax.numpy as jnp
from jax import lax
from jax.experimental import pallas as pl
from jax.experimental.pallas import tpu as pltpu

B, H, D, BS = 8, 8, 128, 16
NB_LOCAL = 512
NZ = 4
CHUNK_PAGES = 64
NCHUNK = NB_LOCAL // CHUNK_PAGES
NDMA = 2 * NCHUNK
NSLOT = 4
AHEAD = 3
CHUNK_TOK = CHUNK_PAGES * BS
T_LOCAL = NB_LOCAL * BS
NEG = -1e30
SCALE = D ** -0.5


def _body(q_ref, bt_ref, lens_ref, k_hbm, v_hbm, out_ref,
          kv_buf, dma_sems, s_ref, part_ref, comm_ref, send_sems, recv_sems):
    my_x = lax.axis_index("x")
    my_y = lax.axis_index("y")
    my_z = lax.axis_index("z")
    z_off = my_z * NB_LOCAL

    def chunk_dma(d):
        src = k_hbm if d < NCHUNK else v_hbm
        c = d % NCHUNK
        return pltpu.make_async_copy(
            src.at[pl.ds(c * CHUNK_PAGES, CHUNK_PAGES)],
            kv_buf.at[d % NSLOT],
            dma_sems.at[d % NSLOT],
        )

    for d in range(AHEAD):
        chunk_dma(d).start()

    q = (q_ref[...] * SCALE).astype(jnp.float32)

    m = None
    ell = None
    o_acc = [jnp.zeros((B, D), jnp.float32) for _ in range(H)]

    for d in range(NDMA):
        if d + AHEAD < NDMA:
            chunk_dma(d + AHEAD).start()

        if d == NCHUNK:
            bt_v = bt_ref[...]
            lens_v = lens_ref[...]
            js = lax.broadcasted_iota(jnp.int32, (B, 1, NB_LOCAL), 2)
            valid = js < lens_v[:, :, None]
            pids = lax.broadcasted_iota(
                jnp.int32, (B, NB_LOCAL, 1), 1) + z_off
            eq = bt_v[:, None, :] == pids
            cnt = jnp.sum(
                jnp.where(jnp.logical_and(eq, valid), 1.0, 0.0),
                axis=2,
            )
            p_iota = lax.broadcasted_iota(jnp.int32, (NB_LOCAL, T_LOCAL), 0)
            t_page = lax.broadcasted_iota(
                jnp.int32, (NB_LOCAL, T_LOCAL), 1) // BS
            expand = jnp.where(p_iota == t_page, 1.0, 0.0)
            ctok = lax.dot_general(
                cnt, expand, (((1,), (0,)), ((), ())),
                preferred_element_type=jnp.float32,
            )

            s_all = s_ref[...]
            ctok3 = ctok[:, None, :]
            s_m = jnp.where(ctok3 > 0, s_all, NEG)
            m = jnp.max(s_m, axis=-1)
            p = ctok3 * jnp.exp(s_m - m[:, :, None])
            ell = jnp.sum(p, axis=-1)
            s_ref[...] = p

        chunk_dma(d).wait()
        kc = kv_buf[d % NSLOT].reshape(CHUNK_TOK, H, D)
        c = d % NCHUNK
        if d < NCHUNK:
            for h in range(H):
                sh = lax.dot_general(
                    q[:, h, :], kc[:, h, :], (((1,), (1,)), ((), ())),
                    preferred_element_type=jnp.float32,
                )
                s_ref[:, h, pl.ds(c * CHUNK_TOK, CHUNK_TOK)] = sh
        else:
            for h in range(H):
                ph = s_ref[:, h, pl.ds(c * CHUNK_TOK, CHUNK_TOK)]
                o_acc[h] = o_acc[h] + lax.dot_general(
                    ph, kc[:, h, :], (((0,), (0,)), ((), ())).__class__(
                        (((1,), (0,)), ((), ()))
                    ),
                    preferred_element_type=jnp.float32,
                )

    for h in range(H):
        part_ref[0, :, h, :] = o_acc[h]
    part_ref[1, :, :, :] = jnp.broadcast_to(m[:, :, None], (B, H, D))
    part_ref[2, :, :, :] = jnp.broadcast_to(ell[:, :, None], (B, H, D))

    rdmas = []
    for dd in range(1, NZ):
        tz = lax.rem(my_z + dd, NZ)
        r = pltpu.make_async_remote_copy(
            src_ref=part_ref,
            dst_ref=comm_ref.at[dd - 1],
            send_sem=send_sems.at[dd - 1],
            recv_sem=recv_sems.at[dd - 1],
            device_id=(my_x, my_y, tz),
            device_id_type=pl.DeviceIdType.MESH,
        )
        r.start()
        rdmas.append(r)
    for r in rdmas:
        r.wait_recv()

    bufs = [part_ref[...]] + [comm_ref[i] for i in range(NZ - 1)]
    m_g = bufs[0][1]
    for b in bufs[1:]:
        m_g = jnp.maximum(m_g, b[1])
    num = jnp.zeros((B, H, D), jnp.float32)
    den = jnp.zeros((B, H, D), jnp.float32)
    for b in bufs:
        alpha = jnp.exp(b[1] - m_g)
        num = num + alpha * b[0]
        den = den + alpha * b[2]
    out_ref[...] = (num / den).reshape(B, 1, H, D)

    for r in rdmas:
        r.wait_send()


def kernel(Q, K, V, bt, lens):
    qr = Q.reshape(B, H, D)
    lens_r = lens.reshape(B, 1)
    return pl.pallas_call(
        _body,
        out_shape=jax.ShapeDtypeStruct((B, 1, H, D), jnp.float32),
        in_specs=[
            pl.BlockSpec(memory_space=pltpu.VMEM),
            pl.BlockSpec(memory_space=pltpu.VMEM),
            pl.BlockSpec(memory_space=pltpu.VMEM),
            pl.BlockSpec(memory_space=pltpu.ANY),
            pl.BlockSpec(memory_space=pltpu.ANY),
        ],
        out_specs=pl.BlockSpec(memory_space=pltpu.VMEM),
        scratch_shapes=[
            pltpu.VMEM((NSLOT, CHUNK_PAGES, BS, H, D), jnp.float32),
            pltpu.SemaphoreType.DMA((NSLOT,)),
            pltpu.VMEM((B, H, T_LOCAL), jnp.float32),
            pltpu.VMEM((3, B, H, D), jnp.float32),
            pltpu.VMEM((NZ - 1, 3, B, H, D), jnp.float32),
            pltpu.SemaphoreType.DMA((NZ - 1,)),
            pltpu.SemaphoreType.DMA((NZ - 1,)),
        ],
        compiler_params=pltpu.CompilerParams(
            collective_id=0,
            vmem_limit_bytes=64 * 1024 * 1024,
        ),
    )(qr, bt, lens_r, K, V)


# baseline (device time: 73528 ns/iter reference)
import jax
import jax.numpy as jnp
from jax import lax
from jax.experimental import pallas as pl
from jax.experimental.pallas import tpu as pltpu

B, H, D, BS = 8, 8, 128, 16
NB_LOCAL = 512
NZ = 4
CHUNK_PAGES = 64
NCHUNK = NB_LOCAL // CHUNK_PAGES
NDMA = 2 * NCHUNK
NSLOT = 4
AHEAD = 3
CHUNK_TOK = CHUNK_PAGES * BS
T_LOCAL = NB_LOCAL * BS
NEG = -1e30
SCALE = D ** -0.5


def _body(q_ref, bt_ref, lens_ref, k_hbm, v_hbm, out_ref,
          kv_buf, dma_sems, s_ref, part_ref, comm_ref, send_sems, recv_sems):
    my_x = lax.axis_index("x")
    my_y = lax.axis_index("y")
    my_z = lax.axis_index("z")
    z_off = my_z * NB_LOCAL

    def chunk_dma(d):
        src = k_hbm if d < NCHUNK else v_hbm
        c = d % NCHUNK
        return pltpu.make_async_copy(
            src.at[pl.ds(c * CHUNK_PAGES, CHUNK_PAGES)],
            kv_buf.at[d % NSLOT],
            dma_sems.at[d % NSLOT],
        )

    for d in range(AHEAD):
        chunk_dma(d).start()

    q = (q_ref[...] * SCALE).astype(jnp.float32)

    m = None
    ell = None
    o_acc = [jnp.zeros((B, D), jnp.float32) for _ in range(H)]

    for d in range(NDMA):
        if d + AHEAD < NDMA:
            chunk_dma(d + AHEAD).start()

        if d == NCHUNK:
            bt_v = bt_ref[...]
            lens_v = lens_ref[...]
            js = lax.broadcasted_iota(jnp.int32, (B, 1, NB_LOCAL), 2)
            valid = js < lens_v[:, :, None]
            pids = lax.broadcasted_iota(
                jnp.int32, (B, NB_LOCAL, 1), 1) + z_off
            eq = bt_v[:, None, :] == pids
            cnt = jnp.sum(
                jnp.where(jnp.logical_and(eq, valid), 1.0, 0.0),
                axis=2,
            )
            p_iota = lax.broadcasted_iota(jnp.int32, (NB_LOCAL, T_LOCAL), 0)
            t_page = lax.broadcasted_iota(
                jnp.int32, (NB_LOCAL, T_LOCAL), 1) // BS
            expand = jnp.where(p_iota == t_page, 1.0, 0.0)
            ctok = lax.dot_general(
                cnt, expand, (((1,), (0,)), ((), ())),
                preferred_element_type=jnp.float32,
            )

            s_all = s_ref[...]
            ctok3 = ctok[:, None, :]
            s_m = jnp.where(ctok3 > 0, s_all, NEG)
            m = jnp.max(s_m, axis=-1)
            p = ctok3 * jnp.exp(s_m - m[:, :, None])
            ell = jnp.sum(p, axis=-1)
            s_ref[...] = p

        chunk_dma(d).wait()
        kc = kv_buf[d % NSLOT].reshape(CHUNK_TOK, H, D)
        c = d % NCHUNK
        if d < NCHUNK:
            for h in range(H):
                sh = lax.dot_general(
                    q[:, h, :], kc[:, h, :], (((1,), (1,)), ((), ())),
                    preferred_element_type=jnp.float32,
                )
                s_ref[:, h, pl.ds(c * CHUNK_TOK, CHUNK_TOK)] = sh
        else:
            for h in range(H):
                ph = s_ref[:, h, pl.ds(c * CHUNK_TOK, CHUNK_TOK)]
                o_acc[h] = o_acc[h] + lax.dot_general(
                    ph, kc[:, h, :], (((1,), (0,)), ((), ())),
                    preferred_element_type=jnp.float32,
                )

    for h in range(H):
        part_ref[0, :, h, :] = o_acc[h]
    part_ref[1, :, :, :] = jnp.broadcast_to(m[:, :, None], (B, H, D))
    part_ref[2, :, :, :] = jnp.broadcast_to(ell[:, :, None], (B, H, D))

    rdmas = []
    for dd in range(1, NZ):
        tz = lax.rem(my_z + dd, NZ)
        r = pltpu.make_async_remote_copy(
            src_ref=part_ref,
            dst_ref=comm_ref.at[dd - 1],
            send_sem=send_sems.at[dd - 1],
            recv_sem=recv_sems.at[dd - 1],
            device_id=(my_x, my_y, tz),
            device_id_type=pl.DeviceIdType.MESH,
        )
        r.start()
        rdmas.append(r)
    for r in rdmas:
        r.wait_recv()

    bufs = [part_ref[...]] + [comm_ref[i] for i in range(NZ - 1)]
    m_g = bufs[0][1]
    for b in bufs[1:]:
        m_g = jnp.maximum(m_g, b[1])
    num = jnp.zeros((B, H, D), jnp.float32)
    den = jnp.zeros((B, H, D), jnp.float32)
    for b in bufs:
        alpha = jnp.exp(b[1] - m_g)
        num = num + alpha * b[0]
        den = den + alpha * b[2]
    out_ref[...] = (num / den).reshape(B, 1, H, D)

    for r in rdmas:
        r.wait_send()


def kernel(Q, K, V, bt, lens):
    qr = Q.reshape(B, H, D)
    lens_r = lens.reshape(B, 1)
    return pl.pallas_call(
        _body,
        out_shape=jax.ShapeDtypeStruct((B, 1, H, D), jnp.float32),
        in_specs=[
            pl.BlockSpec(memory_space=pltpu.VMEM),
            pl.BlockSpec(memory_space=pltpu.VMEM),
            pl.BlockSpec(memory_space=pltpu.VMEM),
            pl.BlockSpec(memory_space=pl.ANY),
            pl.BlockSpec(memory_space=pl.ANY),
        ],
        out_specs=pl.BlockSpec(memory_space=pltpu.VMEM),
        scratch_shapes=[
            pltpu.VMEM((NSLOT, CHUNK_PAGES, BS, H, D), jnp.float32),
            pltpu.SemaphoreType.DMA((NSLOT,)),
            pltpu.VMEM((B, H, T_LOCAL), jnp.float32),
            pltpu.VMEM((3, B, H, D), jnp.float32),
            pltpu.VMEM((NZ - 1, 3, B, H, D), jnp.float32),
            pltpu.SemaphoreType.DMA((NZ - 1,)),
            pltpu.SemaphoreType.DMA((NZ - 1,)),
        ],
        compiler_params=pltpu.CompilerParams(
            vmem_limit_bytes=64 * 1024 * 1024,
        ),
    )(qr, bt, lens_r, K, V)


# device time: 73140 ns/iter; 1.0053x vs baseline; 1.0053x over previous
import jax
import jax.numpy as jnp
from jax import lax
from jax.experimental import pallas as pl
from jax.experimental.pallas import tpu as pltpu

B, H, D, BS = 8, 8, 128, 16
NB_LOCAL = 512
NZ = 4
CHUNK_PAGES = 64
NCHUNK = NB_LOCAL // CHUNK_PAGES
NDMA = 2 * NCHUNK
NSLOT = 4
AHEAD = 3
CHUNK_TOK = CHUNK_PAGES * BS
T_LOCAL = NB_LOCAL * BS
NEG = -1e30
SCALE = D ** -0.5


def _body(q_ref, bt_ref, lens_ref, k_hbm, v_hbm, out_ref,
          kv_buf, dma_sems, s_ref, part_ref, comm_ref, send_sems, recv_sems):
    my_x = lax.axis_index("x")
    my_y = lax.axis_index("y")
    my_z = lax.axis_index("z")
    z_off = my_z * NB_LOCAL

    def chunk_dma(d):
        src = k_hbm if d < NCHUNK else v_hbm
        c = d % NCHUNK
        return pltpu.make_async_copy(
            src.at[pl.ds(c * CHUNK_PAGES, CHUNK_PAGES)],
            kv_buf.at[d % NSLOT],
            dma_sems.at[d % NSLOT],
        )

    for d in range(AHEAD):
        chunk_dma(d).start()

    q = (q_ref[...] * SCALE).astype(jnp.float32)

    m = None
    ell = None
    o_acc = [jnp.zeros((B, D), jnp.float32) for _ in range(H)]

    for d in range(NDMA):
        if d + AHEAD < NDMA:
            chunk_dma(d + AHEAD).start()

        if d == NCHUNK:
          with jax.named_scope("softmax"):
            bt_v = bt_ref[...]
            lens_v = lens_ref[...]
            js = lax.broadcasted_iota(jnp.int32, (B, 1, NB_LOCAL), 2)
            valid = js < lens_v[:, :, None]
            pids = lax.broadcasted_iota(
                jnp.int32, (B, NB_LOCAL, 1), 1) + z_off
            eq = bt_v[:, None, :] == pids
            cnt = jnp.sum(
                jnp.where(jnp.logical_and(eq, valid), 1.0, 0.0),
                axis=2,
            )
            ctok = jnp.repeat(cnt, BS, axis=1)

            s_all = s_ref[...]
            ctok3 = ctok[:, None, :]
            s_m = jnp.where(ctok3 > 0, s_all, NEG)
            m = jnp.max(s_m, axis=-1)
            p = ctok3 * jnp.exp(s_m - m[:, :, None])
            ell = jnp.sum(p, axis=-1)
            s_ref[...] = p

        with jax.named_scope(f"dma_wait#d={d}"):
            chunk_dma(d).wait()
        kc = kv_buf[d % NSLOT].reshape(CHUNK_TOK, H, D)
        c = d % NCHUNK
        if d < NCHUNK:
          with jax.named_scope(f"qk#d={d}"):
            for h in range(H):
                sh = lax.dot_general(
                    q[:, h, :], kc[:, h, :], (((1,), (1,)), ((), ())),
                    preferred_element_type=jnp.float32,
                )
                s_ref[:, h, pl.ds(c * CHUNK_TOK, CHUNK_TOK)] = sh
        else:
          with jax.named_scope(f"pv#d={d}"):
            for h in range(0, H, 2):
                ph = s_ref[:, h, pl.ds(c * CHUNK_TOK, CHUNK_TOK)]
                ph1 = s_ref[:, h + 1, pl.ds(c * CHUNK_TOK, CHUNK_TOK)]
                ppair = jnp.concatenate([ph, ph1], axis=0)
                vpair = jnp.concatenate(
                    [kc[:, h, :], kc[:, h + 1, :]], axis=1)
                opair = lax.dot_general(
                    ppair, vpair, (((1,), (0,)), ((), ())),
                    preferred_element_type=jnp.float32,
                )
                o_acc[h] = o_acc[h] + opair[:B, :D]
                o_acc[h + 1] = o_acc[h + 1] + opair[B:, D:]

    with jax.named_scope("pack"):
        for h in range(H):
            part_ref[0, :, h, :] = o_acc[h]
        part_ref[1, :, :, :] = jnp.broadcast_to(m[:, :, None], (B, H, D))
        part_ref[2, :, :, :] = jnp.broadcast_to(ell[:, :, None], (B, H, D))

    rdmas = []
    with jax.named_scope("rdma_send"):
        for dd in range(1, NZ):
            tz = lax.rem(my_z + dd, NZ)
            r = pltpu.make_async_remote_copy(
                src_ref=part_ref,
                dst_ref=comm_ref.at[dd - 1],
                send_sem=send_sems.at[dd - 1],
                recv_sem=recv_sems.at[dd - 1],
                device_id=(my_x, my_y, tz),
                device_id_type=pl.DeviceIdType.MESH,
            )
            r.start()
            rdmas.append(r)
    with jax.named_scope("rdma_wait_recv"):
        for r in rdmas:
            r.wait_recv()

    with jax.named_scope("combine"):
        bufs = [part_ref[...]] + [comm_ref[i] for i in range(NZ - 1)]
        m_g = bufs[0][1]
        for b in bufs[1:]:
            m_g = jnp.maximum(m_g, b[1])
        num = jnp.zeros((B, H, D), jnp.float32)
        den = jnp.zeros((B, H, D), jnp.float32)
        for b in bufs:
            alpha = jnp.exp(b[1] - m_g)
            num = num + alpha * b[0]
            den = den + alpha * b[2]
        out_ref[...] = (num / den).reshape(B, 1, H, D)

    with jax.named_scope("rdma_wait_send"):
        for r in rdmas:
            r.wait_send()


def kernel(Q, K, V, bt, lens):
    qr = Q.reshape(B, H, D)
    lens_r = lens.reshape(B, 1)
    return pl.pallas_call(
        _body,
        out_shape=jax.ShapeDtypeStruct((B, 1, H, D), jnp.float32),
        in_specs=[
            pl.BlockSpec(memory_space=pltpu.VMEM),
            pl.BlockSpec(memory_space=pltpu.VMEM),
            pl.BlockSpec(memory_space=pltpu.VMEM),
            pl.BlockSpec(memory_space=pl.ANY),
            pl.BlockSpec(memory_space=pl.ANY),
        ],
        out_specs=pl.BlockSpec(memory_space=pltpu.VMEM),
        scratch_shapes=[
            pltpu.VMEM((NSLOT, CHUNK_PAGES, BS, H, D), jnp.float32),
            pltpu.SemaphoreType.DMA((NSLOT,)),
            pltpu.VMEM((B, H, T_LOCAL), jnp.float32),
            pltpu.VMEM((3, B, H, D), jnp.float32),
            pltpu.VMEM((NZ - 1, 3, B, H, D), jnp.float32),
            pltpu.SemaphoreType.DMA((NZ - 1,)),
            pltpu.SemaphoreType.DMA((NZ - 1,)),
        ],
        compiler_params=pltpu.CompilerParams(
            vmem_limit_bytes=64 * 1024 * 1024,
        ),
    )(qr, bt, lens_r, K, V)


# device time: 40388 ns/iter; 1.8205x vs baseline; 1.8109x over previous
import os

import jax
import jax.numpy as jnp
from jax import lax
from jax.experimental import pallas as pl
from jax.experimental.pallas import tpu as pltpu

ABLATE = set(filter(None, os.environ.get("ABLATE", "").split(",")))

B, H, D, BS = 8, 8, 128, 16
NB_LOCAL = 512
NZ = 4
CHUNK_PAGES = 64
NCHUNK = NB_LOCAL // CHUNK_PAGES
NDMA = 2 * NCHUNK
NSLOT = 4
AHEAD = 3
CHUNK_TOK = CHUNK_PAGES * BS
T_LOCAL = NB_LOCAL * BS
NEG = -1e30
SCALE = D ** -0.5


def _body(q_ref, bt_ref, lens_ref, k_hbm, v_hbm, out_ref,
          kv_buf, dma_sems, s_ref, part_ref, comm_ref, send_sems, recv_sems):
    my_x = lax.axis_index("x")
    my_y = lax.axis_index("y")
    my_z = lax.axis_index("z")
    z_off = my_z * NB_LOCAL

    def chunk_dma(d):
        src = k_hbm if d < NCHUNK else v_hbm
        c = d % NCHUNK
        return pltpu.make_async_copy(
            src.at[pl.ds(c * CHUNK_PAGES, CHUNK_PAGES)],
            kv_buf.at[d % NSLOT],
            dma_sems.at[d % NSLOT],
        )

    for d in range(AHEAD):
        chunk_dma(d).start()

    q = (q_ref[...] * SCALE).astype(jnp.float32)

    m_acc = [jnp.full((B,), NEG, jnp.float32) for _ in range(H)]
    l_acc = [jnp.zeros((B,), jnp.float32) for _ in range(H)]
    o_acc = [jnp.zeros((B, D), jnp.float32) for _ in range(H)]
    ctok = None

    for d in range(NDMA):
        if d + AHEAD < NDMA:
            chunk_dma(d + AHEAD).start()

        if d == NCHUNK:
          with jax.named_scope("counts"):
            if "nosm" in ABLATE:
                ctok = jnp.ones((B, T_LOCAL), jnp.float32)
            else:
                bt_v = bt_ref[...]
                lens_v = lens_ref[...]
                js = lax.broadcasted_iota(jnp.int32, (B, 1, NB_LOCAL), 2)
                valid = js < lens_v[:, :, None]
                pids = lax.broadcasted_iota(
                    jnp.int32, (B, NB_LOCAL, 1), 1) + z_off
                eq = bt_v[:, None, :] == pids
                cnt = jnp.sum(
                    jnp.where(jnp.logical_and(eq, valid), 1.0, 0.0),
                    axis=2,
                )
                ctok = jnp.repeat(cnt, BS, axis=1)

        with jax.named_scope(f"dma_wait#d={d}"):
            chunk_dma(d).wait()
        kc = kv_buf[d % NSLOT].reshape(CHUNK_TOK, H * D)
        c = d % NCHUNK
        if d < NCHUNK:
          if "noqk" in ABLATE:
              continue
          with jax.named_scope(f"qk#d={d}"):
            for h in range(H):
                sh = lax.dot_general(
                    q[:, h, :], kc[:, h * D:(h + 1) * D],
                    (((1,), (1,)), ((), ())),
                    preferred_element_type=jnp.float32,
                )
                s_ref[:, h, pl.ds(c * CHUNK_TOK, CHUNK_TOK)] = sh
                m_acc[h] = jnp.maximum(m_acc[h], jnp.max(sh, axis=1))
        else:
          if "nopv" in ABLATE:
              continue
          with jax.named_scope(f"pv#d={d}"):
            cs = ctok[:, c * CHUNK_TOK:(c + 1) * CHUNK_TOK]
            for h in range(0, H, 2):
                ps = []
                for hh in (h, h + 1):
                    sh = s_ref[:, hh, pl.ds(c * CHUNK_TOK, CHUNK_TOK)]
                    p = cs * jnp.exp(sh - m_acc[hh][:, None])
                    l_acc[hh] = l_acc[hh] + jnp.sum(p, axis=1)
                    ps.append(p)
                ppair = jnp.concatenate(ps, axis=0)
                opair = lax.dot_general(
                    ppair, kc[:, h * D:(h + 2) * D],
                    (((1,), (0,)), ((), ())),
                    preferred_element_type=jnp.float32,
                )
                o_acc[h] = o_acc[h] + opair[:B, :D]
                o_acc[h + 1] = o_acc[h + 1] + opair[B:, D:]

    with jax.named_scope("pack"):
        for h in range(H):
            part_ref[0, :, h, :] = o_acc[h]
            part_ref[1, :, h, :] = jnp.broadcast_to(
                m_acc[h][:, None], (B, D))
            part_ref[2, :, h, :] = jnp.broadcast_to(
                l_acc[h][:, None], (B, D))

    rdmas = []
    with jax.named_scope("rdma_send"):
        for dd in range(1, NZ) if "nordma" not in ABLATE else []:
            tz = lax.rem(my_z + dd, NZ)
            r = pltpu.make_async_remote_copy(
                src_ref=part_ref,
                dst_ref=comm_ref.at[dd - 1],
                send_sem=send_sems.at[dd - 1],
                recv_sem=recv_sems.at[dd - 1],
                device_id=(my_x, my_y, tz),
                device_id_type=pl.DeviceIdType.MESH,
            )
            r.start()
            rdmas.append(r)
    with jax.named_scope("rdma_wait_recv"):
        for r in rdmas:
            r.wait_recv()

    with jax.named_scope("combine"):
        if "nordma" in ABLATE:
            bufs = [part_ref[...]]
        else:
            bufs = [part_ref[...]] + [comm_ref[i] for i in range(NZ - 1)]
        m_g = bufs[0][1]
        for b in bufs[1:]:
            m_g = jnp.maximum(m_g, b[1])
        num = jnp.zeros((B, H, D), jnp.float32)
        den = jnp.zeros((B, H, D), jnp.float32)
        for b in bufs:
            alpha = jnp.exp(b[1] - m_g)
            num = num + alpha * b[0]
            den = den + alpha * b[2]
        out_ref[...] = (num / den).reshape(B, 1, H, D)

    with jax.named_scope("rdma_wait_send"):
        for r in rdmas:
            r.wait_send()


def kernel(Q, K, V, bt, lens):
    qr = Q.reshape(B, H, D)
    lens_r = lens.reshape(B, 1)
    return pl.pallas_call(
        _body,
        out_shape=jax.ShapeDtypeStruct((B, 1, H, D), jnp.float32),
        in_specs=[
            pl.BlockSpec(memory_space=pltpu.VMEM),
            pl.BlockSpec(memory_space=pltpu.VMEM),
            pl.BlockSpec(memory_space=pltpu.VMEM),
            pl.BlockSpec(memory_space=pl.ANY),
            pl.BlockSpec(memory_space=pl.ANY),
        ],
        out_specs=pl.BlockSpec(memory_space=pltpu.VMEM),
        scratch_shapes=[
            pltpu.VMEM((NSLOT, CHUNK_PAGES, BS, H, D), jnp.float32),
            pltpu.SemaphoreType.DMA((NSLOT,)),
            pltpu.VMEM((B, H, T_LOCAL), jnp.float32),
            pltpu.VMEM((3, B, H, D), jnp.float32),
            pltpu.VMEM((NZ - 1, 3, B, H, D), jnp.float32),
            pltpu.SemaphoreType.DMA((NZ - 1,)),
            pltpu.SemaphoreType.DMA((NZ - 1,)),
        ],
        compiler_params=pltpu.CompilerParams(
            vmem_limit_bytes=64 * 1024 * 1024,
        ),
    )(qr, bt, lens_r, K, V)


# device time: 33989 ns/iter; 2.1633x vs baseline; 1.1883x over previous
import os

import jax
import jax.numpy as jnp
from jax import lax
from jax.experimental import pallas as pl
from jax.experimental.pallas import tpu as pltpu

ABLATE = set(filter(None, os.environ.get("ABLATE", "").split(",")))

B, H, D, BS = 8, 8, 128, 16
NB_LOCAL = 512
NZ = 4
CHUNK_PAGES = 64
NCHUNK = NB_LOCAL // CHUNK_PAGES
NDMA = 2 * NCHUNK
NSLOT = 4
AHEAD = 3
CHUNK_TOK = CHUNK_PAGES * BS
T_LOCAL = NB_LOCAL * BS
NEG = -1e30
SCALE = D ** -0.5


def _body(q_ref, bt_ref, lens_ref, k_hbm, v_hbm, out_ref,
          kv_buf, dma_sems, s_ref, part_ref, comm_ref, send_sems, recv_sems):
    my_x = lax.axis_index("x")
    my_y = lax.axis_index("y")
    my_z = lax.axis_index("z")
    z_off = my_z * NB_LOCAL

    def chunk_dma(d):
        src = k_hbm if d < NCHUNK else v_hbm
        c = d % NCHUNK
        return pltpu.make_async_copy(
            src.at[pl.ds(c * CHUNK_PAGES, CHUNK_PAGES)],
            kv_buf.at[d % NSLOT],
            dma_sems.at[d % NSLOT],
        )

    for d in range(AHEAD):
        chunk_dma(d).start()

    barrier_sem = pltpu.get_barrier_semaphore()
    if "nordma" not in ABLATE:
        for dd in range(1, NZ):
            pl.semaphore_signal(
                barrier_sem, inc=1,
                device_id=(my_x, my_y, lax.rem(my_z + dd, NZ)),
                device_id_type=pl.DeviceIdType.MESH,
            )

    q = (q_ref[...] * SCALE).astype(jnp.float32)

    m_acc = [jnp.full((B,), NEG, jnp.float32) for _ in range(H)]
    l_acc = [jnp.zeros((B,), jnp.float32) for _ in range(H)]
    o_acc = [jnp.zeros((B, D), jnp.float32) for _ in range(H)]
    ctok = None

    for d in range(NDMA):
        if d + AHEAD < NDMA:
            chunk_dma(d + AHEAD).start()

        if d == NCHUNK:
          with jax.named_scope("counts"):
            if "nosm" in ABLATE:
                ctok = jnp.ones((B, T_LOCAL), jnp.float32)
            else:
                bt_v = bt_ref[...]
                lens_v = lens_ref[...]
                js = lax.broadcasted_iota(jnp.int32, (B, 1, NB_LOCAL), 2)
                valid = js < lens_v[:, :, None]
                pids = lax.broadcasted_iota(
                    jnp.int32, (B, NB_LOCAL, 1), 1) + z_off
                eq = bt_v[:, None, :] == pids
                cnt = jnp.sum(
                    jnp.where(jnp.logical_and(eq, valid), 1.0, 0.0),
                    axis=2,
                )
                ctok = jnp.repeat(cnt, BS, axis=1)

        with jax.named_scope(f"dma_wait#d={d}"):
            chunk_dma(d).wait()
        kc = kv_buf[d % NSLOT].reshape(CHUNK_TOK, H * D)
        c = d % NCHUNK
        if d < NCHUNK:
          if "noqk" in ABLATE:
              continue
          with jax.named_scope(f"qk#d={d}"):
            for h in range(H):
                sh = lax.dot_general(
                    q[:, h, :], kc[:, h * D:(h + 1) * D],
                    (((1,), (1,)), ((), ())),
                    preferred_element_type=jnp.float32,
                )
                s_ref[:, h, pl.ds(c * CHUNK_TOK, CHUNK_TOK)] = sh
                m_acc[h] = jnp.maximum(m_acc[h], jnp.max(sh, axis=1))
        else:
          if "nopv" in ABLATE:
              continue
          with jax.named_scope(f"pv#d={d}"):
            cs = ctok[:, c * CHUNK_TOK:(c + 1) * CHUNK_TOK]
            for h in range(0, H, 2):
                ps = []
                for hh in (h, h + 1):
                    sh = s_ref[:, hh, pl.ds(c * CHUNK_TOK, CHUNK_TOK)]
                    p = cs * jnp.exp(sh - m_acc[hh][:, None])
                    l_acc[hh] = l_acc[hh] + jnp.sum(p, axis=1)
                    ps.append(p)
                ppair = jnp.concatenate(ps, axis=0)
                opair = lax.dot_general(
                    ppair, kc[:, h * D:(h + 2) * D],
                    (((1,), (0,)), ((), ())),
                    preferred_element_type=jnp.float32,
                )
                o_acc[h] = o_acc[h] + opair[:B, :D]
                o_acc[h + 1] = o_acc[h + 1] + opair[B:, D:]

    with jax.named_scope("pack"):
        for h in range(H):
            part_ref[0, :, h, :] = o_acc[h]
            part_ref[1, :, h, :] = jnp.broadcast_to(
                m_acc[h][:, None], (B, D))
            part_ref[2, :, h, :] = jnp.broadcast_to(
                l_acc[h][:, None], (B, D))

    rdmas = []
    with jax.named_scope("rdma_send"):
        if "nordma" not in ABLATE:
            pl.semaphore_wait(barrier_sem, NZ - 1)
        for dd in range(1, NZ) if "nordma" not in ABLATE else []:
            tz = lax.rem(my_z + dd, NZ)
            r = pltpu.make_async_remote_copy(
                src_ref=part_ref,
                dst_ref=comm_ref.at[dd - 1],
                send_sem=send_sems.at[dd - 1],
                recv_sem=recv_sems.at[dd - 1],
                device_id=(my_x, my_y, tz),
                device_id_type=pl.DeviceIdType.MESH,
            )
            r.start()
            rdmas.append(r)
    with jax.named_scope("rdma_wait_recv"):
        for r in rdmas:
            r.wait_recv()

    with jax.named_scope("combine"):
        if "nordma" in ABLATE:
            bufs = [part_ref[...]]
        else:
            bufs = [part_ref[...]] + [comm_ref[i] for i in range(NZ - 1)]
        m_g = bufs[0][1]
        for b in bufs[1:]:
            m_g = jnp.maximum(m_g, b[1])
        num = jnp.zeros((B, H, D), jnp.float32)
        den = jnp.zeros((B, H, D), jnp.float32)
        for b in bufs:
            alpha = jnp.exp(b[1] - m_g)
            num = num + alpha * b[0]
            den = den + alpha * b[2]
        out_ref[...] = (num / den).reshape(B, 1, H, D)

    with jax.named_scope("rdma_wait_send"):
        for r in rdmas:
            r.wait_send()


def kernel(Q, K, V, bt, lens):
    qr = Q.reshape(B, H, D)
    lens_r = lens.reshape(B, 1)
    return pl.pallas_call(
        _body,
        out_shape=jax.ShapeDtypeStruct((B, 1, H, D), jnp.float32),
        in_specs=[
            pl.BlockSpec(memory_space=pltpu.VMEM),
            pl.BlockSpec(memory_space=pltpu.VMEM),
            pl.BlockSpec(memory_space=pltpu.VMEM),
            pl.BlockSpec(memory_space=pl.ANY),
            pl.BlockSpec(memory_space=pl.ANY),
        ],
        out_specs=pl.BlockSpec(memory_space=pltpu.VMEM),
        scratch_shapes=[
            pltpu.VMEM((NSLOT, CHUNK_PAGES, BS, H, D), jnp.float32),
            pltpu.SemaphoreType.DMA((NSLOT,)),
            pltpu.VMEM((B, H, T_LOCAL), jnp.float32),
            pltpu.VMEM((3, B, H, D), jnp.float32),
            pltpu.VMEM((NZ - 1, 3, B, H, D), jnp.float32),
            pltpu.SemaphoreType.DMA((NZ - 1,)),
            pltpu.SemaphoreType.DMA((NZ - 1,)),
        ],
        compiler_params=pltpu.CompilerParams(
            collective_id=0,
            vmem_limit_bytes=64 * 1024 * 1024,
        ),
    )(qr, bt, lens_r, K, V)


# device time: 32781 ns/iter; 2.2430x vs baseline; 1.0369x over previous
import os

import jax
import jax.numpy as jnp
from jax import lax
from jax.experimental import pallas as pl
from jax.experimental.pallas import tpu as pltpu

ABLATE = set(filter(None, os.environ.get("ABLATE", "").split(",")))

B, H, D, BS = 8, 8, 128, 16
NB_LOCAL = 512
NZ = 4
CHUNK_PAGES = 64
NCHUNK = NB_LOCAL // CHUNK_PAGES
NDMA = 2 * NCHUNK
NSLOT = 4
AHEAD = 3
CHUNK_TOK = CHUNK_PAGES * BS
T_LOCAL = NB_LOCAL * BS
NEG = -1e30
SCALE = D ** -0.5


def _body(q_ref, bt_ref, lens_ref, k_hbm, v_hbm, out_ref,
          kv_buf, dma_sems, s_ref, part_ref, comm_ref, send_sems, recv_sems):
    my_x = lax.axis_index("x")
    my_y = lax.axis_index("y")
    my_z = lax.axis_index("z")
    z_off = my_z * NB_LOCAL

    def chunk_dma(d):
        src = k_hbm if d < NCHUNK else v_hbm
        c = d % NCHUNK
        return pltpu.make_async_copy(
            src.at[pl.ds(c * CHUNK_PAGES, CHUNK_PAGES)],
            kv_buf.at[d % NSLOT],
            dma_sems.at[d % NSLOT],
        )

    for d in range(AHEAD):
        chunk_dma(d).start()

    if "nordma" not in ABLATE:
        barrier_sem = pltpu.get_barrier_semaphore()
        for dd in range(1, NZ):
            pl.semaphore_signal(
                barrier_sem, inc=1,
                device_id=(my_x, my_y, lax.rem(my_z + dd, NZ)),
                device_id_type=pl.DeviceIdType.MESH,
            )

    q = (q_ref[...] * SCALE).astype(jnp.float32)

    m_acc = [jnp.full((B,), NEG, jnp.float32) for _ in range(H)]
    l_acc = [jnp.zeros((B,), jnp.float32) for _ in range(H)]
    o_acc = [jnp.zeros((B, D), jnp.float32) for _ in range(H)]
    ctok = None

    for d in range(NDMA):
        if d + AHEAD < NDMA:
            chunk_dma(d + AHEAD).start()

        if d == NCHUNK:
            m_acc = [
                mh.astype(jnp.bfloat16).astype(jnp.float32) for mh in m_acc
            ]
        if d == NCHUNK:
          with jax.named_scope("counts"):
            if "nosm" in ABLATE:
                ctok = jnp.ones((B, T_LOCAL), jnp.float32)
            else:
                bt_v = bt_ref[...]
                lens_v = lens_ref[...]
                js = lax.broadcasted_iota(jnp.int32, (B, 1, NB_LOCAL), 2)
                valid = js < lens_v[:, :, None]
                pids = lax.broadcasted_iota(
                    jnp.int32, (B, NB_LOCAL, 1), 1) + z_off
                eq = bt_v[:, None, :] == pids
                cnt = jnp.sum(
                    jnp.where(jnp.logical_and(eq, valid), 1.0, 0.0),
                    axis=2,
                )
                ctok = jnp.repeat(cnt, BS, axis=1)

        with jax.named_scope(f"dma_wait#d={d}"):
            chunk_dma(d).wait()
        kc = kv_buf[d % NSLOT].reshape(CHUNK_TOK, H * D)
        c = d % NCHUNK
        if d < NCHUNK:
          if "noqk" in ABLATE:
              continue
          with jax.named_scope(f"qk#d={d}"):
            for h in range(H):
                sh = lax.dot_general(
                    q[:, h, :], kc[:, h * D:(h + 1) * D],
                    (((1,), (1,)), ((), ())),
                    preferred_element_type=jnp.float32,
                )
                s_ref[:, h, pl.ds(c * CHUNK_TOK, CHUNK_TOK)] = sh
                m_acc[h] = jnp.maximum(m_acc[h], jnp.max(sh, axis=1))
        else:
          if "nopv" in ABLATE:
              continue
          with jax.named_scope(f"pv#d={d}"):
            cs = ctok[:, c * CHUNK_TOK:(c + 1) * CHUNK_TOK]
            for h in range(0, H, 2):
                ps = []
                for hh in (h, h + 1):
                    sh = s_ref[:, hh, pl.ds(c * CHUNK_TOK, CHUNK_TOK)]
                    p = cs * jnp.exp(sh - m_acc[hh][:, None])
                    l_acc[hh] = l_acc[hh] + jnp.sum(p, axis=1)
                    ps.append(p)
                ppair = jnp.concatenate(ps, axis=0)
                opair = lax.dot_general(
                    ppair, kc[:, h * D:(h + 2) * D],
                    (((1,), (0,)), ((), ())),
                    preferred_element_type=jnp.float32,
                )
                o_acc[h] = o_acc[h] + opair[:B, :D]
                o_acc[h + 1] = o_acc[h + 1] + opair[B:, D:]

    with jax.named_scope("pack"):
        for h in range(H):
            part_ref[0, :, h, :] = o_acc[h].astype(jnp.bfloat16)
            part_ref[1, :, h, :] = jnp.broadcast_to(
                m_acc[h][:, None], (B, D)).astype(jnp.bfloat16)
            part_ref[2, :, h, :] = jnp.broadcast_to(
                l_acc[h][:, None], (B, D)).astype(jnp.bfloat16)

    rdmas = []
    with jax.named_scope("rdma_send"):
        if "nordma" not in ABLATE:
            pl.semaphore_wait(barrier_sem, NZ - 1)
        for dd in range(1, NZ) if "nordma" not in ABLATE else []:
            tz = lax.rem(my_z + dd, NZ)
            r = pltpu.make_async_remote_copy(
                src_ref=part_ref,
                dst_ref=comm_ref.at[dd - 1],
                send_sem=send_sems.at[dd - 1],
                recv_sem=recv_sems.at[dd - 1],
                device_id=(my_x, my_y, tz),
                device_id_type=pl.DeviceIdType.MESH,
            )
            r.start()
            rdmas.append(r)
    with jax.named_scope("rdma_wait_recv"):
        for r in rdmas:
            r.wait_recv()

    with jax.named_scope("combine"):
        if "nordma" in ABLATE:
            bufs = [part_ref[...].astype(jnp.float32)]
        else:
            bufs = [part_ref[...].astype(jnp.float32)] + [
                comm_ref[i].astype(jnp.float32) for i in range(NZ - 1)
            ]
        m_g = bufs[0][1]
        for b in bufs[1:]:
            m_g = jnp.maximum(m_g, b[1])
        num = jnp.zeros((B, H, D), jnp.float32)
        den = jnp.zeros((B, H, D), jnp.float32)
        for b in bufs:
            alpha = jnp.exp(b[1] - m_g)
            num = num + alpha * b[0]
            den = den + alpha * b[2]
        out_ref[...] = (num / den).reshape(B, 1, H, D)

    with jax.named_scope("rdma_wait_send"):
        for r in rdmas:
            r.wait_send()


def kernel(Q, K, V, bt, lens):
    qr = Q.reshape(B, H, D)
    lens_r = lens.reshape(B, 1)
    return pl.pallas_call(
        _body,
        out_shape=jax.ShapeDtypeStruct((B, 1, H, D), jnp.float32),
        in_specs=[
            pl.BlockSpec(memory_space=pltpu.VMEM),
            pl.BlockSpec(memory_space=pltpu.VMEM),
            pl.BlockSpec(memory_space=pltpu.VMEM),
            pl.BlockSpec(memory_space=pl.ANY),
            pl.BlockSpec(memory_space=pl.ANY),
        ],
        out_specs=pl.BlockSpec(memory_space=pltpu.VMEM),
        scratch_shapes=[
            pltpu.VMEM((NSLOT, CHUNK_PAGES, BS, H, D), jnp.float32),
            pltpu.SemaphoreType.DMA((NSLOT,)),
            pltpu.VMEM((B, H, T_LOCAL), jnp.float32),
            pltpu.VMEM((3, B, H, D), jnp.bfloat16),
            pltpu.VMEM((NZ - 1, 3, B, H, D), jnp.bfloat16),
            pltpu.SemaphoreType.DMA((NZ - 1,)),
            pltpu.SemaphoreType.DMA((NZ - 1,)),
        ],
        compiler_params=pltpu.CompilerParams(
            collective_id=None if "nordma" in ABLATE else 0,
            vmem_limit_bytes=64 * 1024 * 1024,
        ),
    )(qr, bt, lens_r, K, V)
